# baseline (speedup 1.0000x reference)
import jax
import jax.numpy as jnp
from jax.experimental import pallas as pl
from jax.experimental.pallas import tpu as pltpu

_S3 = 0.5773502691896258


def _tp_kernel(x_ref, y_ref, o_ref):
    x = x_ref[...]
    y = y_ref[...]
    n, t = x.shape[0], x.shape[1]

    ci = jax.lax.broadcasted_iota(jnp.int32, (1, 1, 4, 1), 2)
    cvec = jnp.where(ci == 0, 1.0, _S3).astype(x.dtype)
    o_ref[:, :, 0:4, :] = x[:, :, 0:1, :] * y * cvec
    xs = x[:, :, 1:4, :]
    o_ref[:, :, 4:7, :] = (_S3 * xs) * y[:, :, 0:1, :]
    q = xs * y[:, :, 1:4, :]
    d = q[:, :, 0, :] + q[:, :, 1, :] + q[:, :, 2, :]
    o_ref[:, :, 7, :] = _S3 * d


@jax.jit
def kernel(x, y):
    E, C, D = x.shape
    ET = E // 128

    def to_slabs(a):
        return a.reshape(ET, 128, C, D).transpose(2, 0, 3, 1)

    xv = to_slabs(x)
    yv = to_slabs(y)

    tb = next(d for d in (25, 10, 5, 2, 1) if ET % d == 0)
    out = pl.pallas_call(
        _tp_kernel,
        grid=(ET // tb,),
        in_specs=[
            pl.BlockSpec((C, tb, D, 128), lambda i: (0, i, 0, 0)),
            pl.BlockSpec((C, tb, D, 128), lambda i: (0, i, 0, 0)),
        ],
        out_specs=pl.BlockSpec((C, tb, 2 * D, 128), lambda i: (0, i, 0, 0)),
        out_shape=jax.ShapeDtypeStruct((C, ET, 2 * D, 128), x.dtype),
        compiler_params=pltpu.CompilerParams(
            dimension_semantics=("arbitrary",),
        ),
    )(xv, yv)

    return (out.transpose(1, 3, 0, 2)
               .reshape(E, C, 2 * D))
